# baseline (speedup 1.0000x reference)
import jax
import jax.numpy as jnp
from jax.experimental import pallas as pl


def _push_kernel(x_ref, s_ref, o_ref):
    o_ref[:, :, 1:] = s_ref[:, :, :-1]
    o_ref[:, :, 0] = x_ref[...]


def kernel(x, state):
    B, N, H = state.shape
    bb, nb = 8, 1024
    grid = (B // bb, N // nb)
    return pl.pallas_call(
        _push_kernel,
        grid=grid,
        in_specs=[
            pl.BlockSpec((bb, nb), lambda i, j: (i, j)),
            pl.BlockSpec((bb, nb, H), lambda i, j: (i, j, 0)),
        ],
        out_specs=pl.BlockSpec((bb, nb, H), lambda i, j: (i, j, 0)),
        out_shape=jax.ShapeDtypeStruct((B, N, H), state.dtype),
    )(x, state)


# speedup vs baseline: 2.2194x; 2.2194x over previous
import jax
import jax.numpy as jnp
from jax import lax
from jax.experimental import pallas as pl

_H = 8


def _push_kernel(x_ref, s_ref, o_ref):
    s = s_ref[...]
    rolled = jnp.roll(s, 1, axis=1)
    xr = jnp.repeat(x_ref[...], _H, axis=1)
    lane = lax.broadcasted_iota(jnp.int32, s.shape, 1)
    o_ref[...] = jnp.where(lane % _H == 0, xr, rolled)


def kernel(x, state):
    B, N, H = state.shape
    L = N * H
    sf = state.reshape(B, L)
    bb, lb = 16, 4096
    grid = (B // bb, L // lb)
    out = pl.pallas_call(
        _push_kernel,
        grid=grid,
        in_specs=[
            pl.BlockSpec((bb, lb // H), lambda i, j: (i, j)),
            pl.BlockSpec((bb, lb), lambda i, j: (i, j)),
        ],
        out_specs=pl.BlockSpec((bb, lb), lambda i, j: (i, j)),
        out_shape=jax.ShapeDtypeStruct((B, L), state.dtype),
    )(x, sf)
    return out.reshape(B, N, H)


# speedup vs baseline: 5.0902x; 2.2936x over previous
import jax
import jax.numpy as jnp
from jax import lax
from jax.experimental import pallas as pl
from jax.experimental.pallas import tpu as pltpu

_H = 8


def _push_kernel(xv_ref, s_ref, o_ref):
    s = s_ref[...]
    rolled = pltpu.roll(s, 1, axis=2)
    bb, sb, _ = s.shape
    a1 = jnp.repeat(xv_ref[...], 8, axis=1)
    s_i = lax.broadcasted_iota(jnp.int32, s.shape, 1)
    l_i = lax.broadcasted_iota(jnp.int32, s.shape, 2)
    idx = 16 * (s_i % 8) + l_i // _H
    xr = jnp.take_along_axis(a1, idx, axis=2)
    o_ref[...] = jnp.where(l_i % _H == 0, xr, rolled)


def kernel(x, state):
    B, N, H = state.shape
    R = N * H // 128
    sv = state.reshape(B, R, 128)
    xv = x.reshape(B, N // 128, 128)
    bb, sb = 16, 64
    grid = (B // bb, R // sb)
    out = pl.pallas_call(
        _push_kernel,
        grid=grid,
        in_specs=[
            pl.BlockSpec((bb, sb // 8, 128), lambda i, j: (i, j, 0)),
            pl.BlockSpec((bb, sb, 128), lambda i, j: (i, j, 0)),
        ],
        out_specs=pl.BlockSpec((bb, sb, 128), lambda i, j: (i, j, 0)),
        out_shape=jax.ShapeDtypeStruct((B, R, 128), state.dtype),
        compiler_params=pltpu.CompilerParams(
            dimension_semantics=("parallel", "parallel"),
        ),
    )(xv, sv)
    return out.reshape(B, N, H)
